# baseline (speedup 1.0000x reference)
import functools

import jax
import jax.numpy as jnp
from jax import lax
from jax.experimental import pallas as pl
from jax.experimental.pallas import tpu as pltpu
from jax.experimental.pallas import tpu_sc as plsc

N = 10000
E = 320000
D = 128
R = 8
NC = 2
NS = 16
NW = NC * NS
CH = 128
CPT = 80
NCH = NW * CPT
EPAD = NCH * CH
NSEG = N * R
DUMP = N
CROW = (N + 240) * R // D
NPAD = N + 240
ACC_PT = NPAD // NS
STG = 5
CPG = CPT // STG

_mesh = plsc.VectorSubcoreMesh(core_axis_name="c", subcore_axis_name="s")
_sc_params = pltpu.CompilerParams(needs_layout_passes=False)


@functools.partial(
    pl.kernel,
    out_type=jax.ShapeDtypeStruct((NW, CROW, D), jnp.float32),
    mesh=_mesh,
    compiler_params=_sc_params,
    scratch_types=[
        pltpu.VMEM((8, CH), jnp.int32),
        pltpu.VMEM((8, CH), jnp.int32),
        pltpu.VMEM((CROW, D), jnp.float32),
    ],
)
def _sc_count(dst_h, et_h,
              cnt_o,
              v_dst, v_et, v_cnt):
    c = lax.axis_index("c")
    s = lax.axis_index("s")
    w = s * NC + c

    def zrow(i, _):
        for q in range(D // 16):
            v_cnt[i, pl.ds(q * 16, 16)] = jnp.zeros((16,), jnp.float32)
        return 0

    lax.fori_loop(0, CROW, zrow, 0)

    ones16 = jnp.ones((16,), jnp.float32)
    lanes = lax.broadcasted_iota(jnp.int32, (16,), 0)

    def group_body(gi, _):
        row = w * CPT + gi * 8
        pltpu.sync_copy(dst_h.at[pl.ds(row, 8)], v_dst)
        pltpu.sync_copy(et_h.at[pl.ds(row, 8)], v_et)
        for k in range(8):
            for g in range(CH // 16):
                sl = pl.ds(g * 16, 16)
                seg16 = v_dst[k, sl] * R + v_et[k, sl]
                r16 = lax.shift_right_logical(seg16, 7)
                c16 = lax.bitwise_and(seg16, 127)
                for lane in range(16):
                    plsc.addupdate_scatter(v_cnt, [r16, c16], ones16,
                                           mask=lanes == lane)
        return 0

    lax.fori_loop(0, CPT // 8, group_body, 0)
    pltpu.sync_copy(v_cnt, cnt_o.at[w])


@functools.partial(
    pl.kernel,
    out_type=(
        jax.ShapeDtypeStruct((NCH, CH), jnp.float32),
        jax.ShapeDtypeStruct((NCH, CH), jnp.int32),
    ),
    mesh=_mesh,
    compiler_params=_sc_params,
    scratch_types=[
        pltpu.VMEM((CROW, D), jnp.float32),
        pltpu.VMEM((8, CH), jnp.int32),
        pltpu.VMEM((8, CH), jnp.int32),
        pltpu.VMEM((8, CH), jnp.int32),
        pltpu.VMEM((8, CH), jnp.float32),
        pltpu.VMEM((8, CH), jnp.int32),
    ],
)
def _sc_scale(src_h, dst_h, et_h, inv_h,
              scale_o, gidx_o,
              v_inv, v_src, v_dst, v_et, v_scale, v_gidx):
    c = lax.axis_index("c")
    s = lax.axis_index("s")
    w = s * NC + c

    pltpu.sync_copy(inv_h, v_inv)

    def group_body(gi, _):
        row = w * CPT + gi * 8
        pltpu.sync_copy(src_h.at[pl.ds(row, 8)], v_src)
        pltpu.sync_copy(dst_h.at[pl.ds(row, 8)], v_dst)
        pltpu.sync_copy(et_h.at[pl.ds(row, 8)], v_et)
        for k in range(8):
            for g in range(CH // 16):
                sl = pl.ds(g * 16, 16)
                t16 = v_et[k, sl]
                seg16 = v_dst[k, sl] * R + t16
                r16 = lax.shift_right_logical(seg16, 7)
                c16 = lax.bitwise_and(seg16, 127)
                v_scale[k, sl] = plsc.load_gather(v_inv, [r16, c16])
                v_gidx[k, sl] = v_src[k, sl] * R + t16
        pltpu.sync_copy(v_scale, scale_o.at[pl.ds(row, 8)])
        pltpu.sync_copy(v_gidx, gidx_o.at[pl.ds(row, 8)])
        return 0

    lax.fori_loop(0, CPT // 8, group_body, 0)


@functools.partial(
    pl.kernel,
    out_type=jax.ShapeDtypeStruct((NC, NPAD, D), jnp.float32),
    mesh=_mesh,
    compiler_params=_sc_params,
    scratch_types=[
        pltpu.VMEM((CPG, CH), jnp.int32),
        pltpu.VMEM((CPG, CH), jnp.int32),
        pltpu.VMEM((CPG, CH), jnp.float32),
        pltpu.VMEM((CH, D), jnp.float32),
        pltpu.VMEM((CH, D), jnp.float32),
        pltpu.VMEM_SHARED((NPAD, D), jnp.float32),
        pltpu.SemaphoreType.DMA,
        pltpu.SemaphoreType.DMA,
        pltpu.SemaphoreType.DMA,
        pltpu.SemaphoreType.DMA,
        pltpu.SemaphoreType.DMA,
        pltpu.SemaphoreType.DMA,
    ],
)
def _sc_layer(xw_h, gidx_h, dst_h, scale_h, zacc_h,
              acc_o,
              v_gidx, v_dst, v_scale, rows0, rows1, acc_sh,
              sem0a, sem0b, sem1a, sem1b, ssem0, ssem1):
    c = lax.axis_index("c")
    s = lax.axis_index("s")
    w = s * NC + c

    pltpu.sync_copy(zacc_h, acc_sh.at[pl.ds(s * ACC_PT, ACC_PT)])
    plsc.subcore_barrier()

    one16 = jnp.full((16,), 1, jnp.int32)

    def _drain_scatter(buf, ssem):
        pltpu.make_async_copy(buf, acc_sh.at[v_dst.at[0]], ssem).wait()

    HH = CH // 2

    def _fire_gather(j, buf, sga, sgb):
        pltpu.async_copy(xw_h.at[v_gidx.at[j, pl.ds(0, HH)]],
                         buf.at[pl.ds(0, HH)], sga)
        pltpu.async_copy(xw_h.at[v_gidx.at[j, pl.ds(HH, HH)]],
                         buf.at[pl.ds(HH, HH)], sgb)

    def _wait_gather(j, buf, sga, sgb):
        pltpu.make_async_copy(xw_h.at[v_gidx.at[j, pl.ds(0, HH)]],
                              buf.at[pl.ds(0, HH)], sga).wait()
        pltpu.make_async_copy(xw_h.at[v_gidx.at[j, pl.ds(HH, HH)]],
                              buf.at[pl.ds(HH, HH)], sgb).wait()

    for stage in range(STG):
        if stage > 0:
            _drain_scatter(rows0, ssem0)
            _drain_scatter(rows1, ssem1)
        row0_ = w * CPT + stage * CPG
        pltpu.sync_copy(gidx_h.at[pl.ds(row0_, CPG)], v_gidx)
        pltpu.sync_copy(dst_h.at[pl.ds(row0_, CPG)], v_dst)
        pltpu.sync_copy(scale_h.at[pl.ds(row0_, CPG)], v_scale)

        _fire_gather(0, rows0, sem0a, sem0b)

        def pair_body(g, _):
            for off, bufA, semAa, semAb, ssemA, bufB, semBa, semBb, ssemB \
                    in ((0, rows0, sem0a, sem0b, ssem0,
                         rows1, sem1a, sem1b, ssem1),
                        (1, rows1, sem1a, sem1b, ssem1,
                         rows0, sem0a, sem0b, ssem0)):
                j = 2 * g + off
                _wait_gather(j, bufA, semAa, semAb)
                if off == 0:
                    @pl.when(g > 0)
                    def _():
                        _drain_scatter(bufB, ssemB)
                    _fire_gather(j + 1, bufB, semBa, semBb)
                else:
                    @pl.when(g < CPG // 2 - 1)
                    def _():
                        _drain_scatter(bufB, ssemB)
                        _fire_gather(j + 1, bufB, semBa, semBb)

                @plsc.parallel_loop(0, CH, unroll=8)
                def _(i):
                    splat = plsc.load_gather(v_scale,
                                             [one16 * j, one16 * i])
                    for q in range(D // 16):
                        sl = pl.ds(q * 16, 16)
                        bufA[i, sl] = bufA[i, sl] * splat

                pltpu.async_copy(bufA, acc_sh.at[v_dst.at[j]], ssemA,
                                 add=True)
            return 0

        lax.fori_loop(0, CPG // 2, pair_body, 0)

    _drain_scatter(rows0, ssem0)
    _drain_scatter(rows1, ssem1)
    plsc.subcore_barrier()
    pltpu.sync_copy(acc_sh.at[pl.ds(s * ACC_PT, ACC_PT)],
                    acc_o.at[c, pl.ds(s * ACC_PT, ACC_PT)])


_BN = 1000


def _inv_body(c_ref, o_ref):
    total = jnp.sum(c_ref[...], axis=0)
    o_ref[...] = 1.0 / jnp.maximum(total, 1.0)


def _tc_inv(cnt):
    return pl.pallas_call(
        _inv_body,
        out_shape=jax.ShapeDtypeStruct((CROW, D), jnp.float32),
    )(cnt)


def _tform_body(h_ref, w_ref, root_ref, xw_ref, hr_ref):
    h = h_ref[...]
    for r in range(R):
        xw_ref[:, r, :] = jnp.dot(h, w_ref[r], preferred_element_type=jnp.float32)
    hr_ref[...] = jnp.dot(h, root_ref[...], preferred_element_type=jnp.float32)


def _tc_transform(h, W, root):
    return pl.pallas_call(
        _tform_body,
        grid=(N // _BN,),
        in_specs=[
            pl.BlockSpec((_BN, D), lambda i: (i, 0)),
            pl.BlockSpec((R, D, D), lambda i: (0, 0, 0)),
            pl.BlockSpec((D, D), lambda i: (0, 0)),
        ],
        out_specs=[
            pl.BlockSpec((_BN, R, D), lambda i: (i, 0, 0)),
            pl.BlockSpec((_BN, D), lambda i: (i, 0)),
        ],
        out_shape=[
            jax.ShapeDtypeStruct((N, R, D), jnp.float32),
            jax.ShapeDtypeStruct((N, D), jnp.float32),
        ],
    )(h, W, root)


def _mid_body(a_ref, pr_ref, b_ref, w_ref, root_ref, xw_ref, hr_ref):
    h = jax.nn.relu(a_ref[0] + a_ref[1] + pr_ref[...] + b_ref[...])
    for r in range(R):
        xw_ref[:, r, :] = jnp.dot(h, w_ref[r], preferred_element_type=jnp.float32)
    hr_ref[...] = jnp.dot(h, root_ref[...], preferred_element_type=jnp.float32)


def _tc_mid(acc, prevroot, b, W, root):
    return pl.pallas_call(
        _mid_body,
        grid=(N // _BN,),
        in_specs=[
            pl.BlockSpec((NC, _BN, D), lambda i: (0, i, 0)),
            pl.BlockSpec((_BN, D), lambda i: (i, 0)),
            pl.BlockSpec((1, D), lambda i: (0, 0)),
            pl.BlockSpec((R, D, D), lambda i: (0, 0, 0)),
            pl.BlockSpec((D, D), lambda i: (0, 0)),
        ],
        out_specs=[
            pl.BlockSpec((_BN, R, D), lambda i: (i, 0, 0)),
            pl.BlockSpec((_BN, D), lambda i: (i, 0)),
        ],
        out_shape=[
            jax.ShapeDtypeStruct((N, R, D), jnp.float32),
            jax.ShapeDtypeStruct((N, D), jnp.float32),
        ],
    )(acc, prevroot, b, W, root)


def _final_body(a_ref, hr_ref, b_ref, o_ref):
    o_ref[...] = jax.nn.relu(a_ref[0] + a_ref[1] + hr_ref[...] + b_ref[...])


def _tc_final(acc, hroot, b):
    return pl.pallas_call(
        _final_body,
        grid=(N // _BN,),
        in_specs=[
            pl.BlockSpec((NC, _BN, D), lambda i: (0, i, 0)),
            pl.BlockSpec((_BN, D), lambda i: (i, 0)),
            pl.BlockSpec((1, D), lambda i: (0, 0)),
        ],
        out_specs=pl.BlockSpec((_BN, D), lambda i: (i, 0)),
        out_shape=jax.ShapeDtypeStruct((N, D), jnp.float32),
    )(acc, hroot, b)


def kernel(x, edge_index, edge_type, W1, root1, b1, W2, root2, b2):
    pad = EPAD - E

    def _prep_edges(arr, padvals):
        a = jnp.concatenate([arr.astype(jnp.int32), padvals]).reshape(NCH, CH)
        return a.reshape(CPT, NW, CH).transpose(1, 0, 2).reshape(NCH, CH)

    pad_dst = DUMP + (jnp.arange(pad, dtype=jnp.int32) % (NPAD - N))
    src_p = _prep_edges(edge_index[0], jnp.zeros((pad,), jnp.int32))
    dst_p = _prep_edges(edge_index[1], pad_dst)
    et_p = _prep_edges(edge_type, jnp.zeros((pad,), jnp.int32))
    zacc = jnp.zeros((ACC_PT, D), jnp.float32)

    cnt = _sc_count(dst_p, et_p)
    inv = _tc_inv(cnt)
    scale_p, gidx_p = _sc_scale(src_p, dst_p, et_p, inv)

    xw1, xroot1 = _tc_transform(x, W1, root1)
    a1 = _sc_layer(xw1.reshape(NSEG, D), gidx_p, dst_p, scale_p, zacc)
    xw2, hroot2 = _tc_mid(a1[:, :N], xroot1, b1.reshape(1, D), W2, root2)
    a2 = _sc_layer(xw2.reshape(NSEG, D), gidx_p, dst_p, scale_p, zacc)
    return _tc_final(a2[:, :N], hroot2, b2.reshape(1, D))


# speedup vs baseline: 1.0180x; 1.0180x over previous
import functools

import jax
import jax.numpy as jnp
from jax import lax
from jax.experimental import pallas as pl
from jax.experimental.pallas import tpu as pltpu
from jax.experimental.pallas import tpu_sc as plsc

N = 10000
E = 320000
D = 128
R = 8
NC = 2
NS = 16
NW = NC * NS
CH = 128
CPT = 80
NCH = NW * CPT
EPAD = NCH * CH
NSEG = N * R
DUMP = N
CROW = (N + 240) * R // D
NPAD = N + 240
ACC_PT = NPAD // NS
STG = 2
CPG = CPT // STG

_mesh = plsc.VectorSubcoreMesh(core_axis_name="c", subcore_axis_name="s")
_sc_params = pltpu.CompilerParams(needs_layout_passes=False)


@functools.partial(
    pl.kernel,
    out_type=jax.ShapeDtypeStruct((NW, CROW, D), jnp.float32),
    mesh=_mesh,
    compiler_params=_sc_params,
    scratch_types=[
        pltpu.VMEM((8, CH), jnp.int32),
        pltpu.VMEM((8, CH), jnp.int32),
        pltpu.VMEM((CROW, D), jnp.float32),
    ],
)
def _sc_count(dst_h, et_h,
              cnt_o,
              v_dst, v_et, v_cnt):
    c = lax.axis_index("c")
    s = lax.axis_index("s")
    w = s * NC + c

    def zrow(i, _):
        for q in range(D // 16):
            v_cnt[i, pl.ds(q * 16, 16)] = jnp.zeros((16,), jnp.float32)
        return 0

    lax.fori_loop(0, CROW, zrow, 0)

    ones16 = jnp.ones((16,), jnp.float32)
    lanes = lax.broadcasted_iota(jnp.int32, (16,), 0)

    def group_body(gi, _):
        row = w * CPT + gi * 8
        pltpu.sync_copy(dst_h.at[pl.ds(row, 8)], v_dst)
        pltpu.sync_copy(et_h.at[pl.ds(row, 8)], v_et)
        for k in range(8):
            for g in range(CH // 16):
                sl = pl.ds(g * 16, 16)
                seg16 = v_dst[k, sl] * R + v_et[k, sl]
                r16 = lax.shift_right_logical(seg16, 7)
                c16 = lax.bitwise_and(seg16, 127)
                for lane in range(16):
                    plsc.addupdate_scatter(v_cnt, [r16, c16], ones16,
                                           mask=lanes == lane)
        return 0

    lax.fori_loop(0, CPT // 8, group_body, 0)
    pltpu.sync_copy(v_cnt, cnt_o.at[w])


@functools.partial(
    pl.kernel,
    out_type=(
        jax.ShapeDtypeStruct((NCH, CH), jnp.float32),
        jax.ShapeDtypeStruct((NCH, CH), jnp.int32),
    ),
    mesh=_mesh,
    compiler_params=_sc_params,
    scratch_types=[
        pltpu.VMEM((CROW, D), jnp.float32),
        pltpu.VMEM((8, CH), jnp.int32),
        pltpu.VMEM((8, CH), jnp.int32),
        pltpu.VMEM((8, CH), jnp.int32),
        pltpu.VMEM((8, CH), jnp.float32),
        pltpu.VMEM((8, CH), jnp.int32),
    ],
)
def _sc_scale(src_h, dst_h, et_h, inv_h,
              scale_o, gidx_o,
              v_inv, v_src, v_dst, v_et, v_scale, v_gidx):
    c = lax.axis_index("c")
    s = lax.axis_index("s")
    w = s * NC + c

    pltpu.sync_copy(inv_h, v_inv)

    def group_body(gi, _):
        row = w * CPT + gi * 8
        pltpu.sync_copy(src_h.at[pl.ds(row, 8)], v_src)
        pltpu.sync_copy(dst_h.at[pl.ds(row, 8)], v_dst)
        pltpu.sync_copy(et_h.at[pl.ds(row, 8)], v_et)
        for k in range(8):
            for g in range(CH // 16):
                sl = pl.ds(g * 16, 16)
                t16 = v_et[k, sl]
                seg16 = v_dst[k, sl] * R + t16
                r16 = lax.shift_right_logical(seg16, 7)
                c16 = lax.bitwise_and(seg16, 127)
                v_scale[k, sl] = plsc.load_gather(v_inv, [r16, c16])
                v_gidx[k, sl] = v_src[k, sl] * R + t16
        pltpu.sync_copy(v_scale, scale_o.at[pl.ds(row, 8)])
        pltpu.sync_copy(v_gidx, gidx_o.at[pl.ds(row, 8)])
        return 0

    lax.fori_loop(0, CPT // 8, group_body, 0)


@functools.partial(
    pl.kernel,
    out_type=jax.ShapeDtypeStruct((NC, NPAD, D), jnp.float32),
    mesh=_mesh,
    compiler_params=_sc_params,
    scratch_types=[
        pltpu.VMEM((CPG, CH), jnp.int32),
        pltpu.VMEM((CPG, CH), jnp.int32),
        pltpu.VMEM((CPG, CH), jnp.float32),
        pltpu.VMEM((CH, D), jnp.float32),
        pltpu.VMEM((CH, D), jnp.float32),
        pltpu.VMEM_SHARED((NPAD, D), jnp.float32),
        pltpu.SemaphoreType.DMA,
        pltpu.SemaphoreType.DMA,
        pltpu.SemaphoreType.DMA,
        pltpu.SemaphoreType.DMA,
        pltpu.SemaphoreType.DMA,
        pltpu.SemaphoreType.DMA,
    ],
)
def _sc_layer(xw_h, gidx_h, dst_h, scale_h, zacc_h,
              acc_o,
              v_gidx, v_dst, v_scale, rows0, rows1, acc_sh,
              sem0a, sem0b, sem1a, sem1b, ssem0, ssem1):
    c = lax.axis_index("c")
    s = lax.axis_index("s")
    w = s * NC + c

    pltpu.sync_copy(zacc_h, acc_sh.at[pl.ds(s * ACC_PT, ACC_PT)])
    plsc.subcore_barrier()

    one16 = jnp.full((16,), 1, jnp.int32)

    def _drain_scatter(buf, ssem):
        pltpu.make_async_copy(buf, acc_sh.at[v_dst.at[0]], ssem).wait()

    HH = CH // 2

    def _fire_gather(j, buf, sga, sgb):
        pltpu.async_copy(xw_h.at[v_gidx.at[j, pl.ds(0, HH)]],
                         buf.at[pl.ds(0, HH)], sga)
        pltpu.async_copy(xw_h.at[v_gidx.at[j, pl.ds(HH, HH)]],
                         buf.at[pl.ds(HH, HH)], sgb)

    def _wait_gather(j, buf, sga, sgb):
        pltpu.make_async_copy(xw_h.at[v_gidx.at[j, pl.ds(0, HH)]],
                              buf.at[pl.ds(0, HH)], sga).wait()
        pltpu.make_async_copy(xw_h.at[v_gidx.at[j, pl.ds(HH, HH)]],
                              buf.at[pl.ds(HH, HH)], sgb).wait()

    for stage in range(STG):
        if stage > 0:
            _drain_scatter(rows0, ssem0)
            _drain_scatter(rows1, ssem1)
        row0_ = w * CPT + stage * CPG
        pltpu.sync_copy(gidx_h.at[pl.ds(row0_, CPG)], v_gidx)
        pltpu.sync_copy(dst_h.at[pl.ds(row0_, CPG)], v_dst)
        pltpu.sync_copy(scale_h.at[pl.ds(row0_, CPG)], v_scale)

        _fire_gather(0, rows0, sem0a, sem0b)

        def pair_body(g, _):
            for off, bufA, semAa, semAb, ssemA, bufB, semBa, semBb, ssemB \
                    in ((0, rows0, sem0a, sem0b, ssem0,
                         rows1, sem1a, sem1b, ssem1),
                        (1, rows1, sem1a, sem1b, ssem1,
                         rows0, sem0a, sem0b, ssem0)):
                j = 2 * g + off
                _wait_gather(j, bufA, semAa, semAb)
                if off == 0:
                    @pl.when(g > 0)
                    def _():
                        _drain_scatter(bufB, ssemB)
                    _fire_gather(j + 1, bufB, semBa, semBb)
                else:
                    @pl.when(g < CPG // 2 - 1)
                    def _():
                        _drain_scatter(bufB, ssemB)
                        _fire_gather(j + 1, bufB, semBa, semBb)

                @plsc.parallel_loop(0, CH, unroll=8)
                def _(i):
                    splat = plsc.load_gather(v_scale,
                                             [one16 * j, one16 * i])
                    for q in range(D // 16):
                        sl = pl.ds(q * 16, 16)
                        bufA[i, sl] = bufA[i, sl] * splat

                pltpu.async_copy(bufA, acc_sh.at[v_dst.at[j]], ssemA,
                                 add=True)
            return 0

        lax.fori_loop(0, CPG // 2, pair_body, 0)

    _drain_scatter(rows0, ssem0)
    _drain_scatter(rows1, ssem1)
    plsc.subcore_barrier()
    pltpu.sync_copy(acc_sh.at[pl.ds(s * ACC_PT, ACC_PT)],
                    acc_o.at[c, pl.ds(s * ACC_PT, ACC_PT)])


_BN = 1000


def _inv_body(c_ref, o_ref):
    total = jnp.sum(c_ref[...], axis=0)
    o_ref[...] = 1.0 / jnp.maximum(total, 1.0)


def _tc_inv(cnt):
    return pl.pallas_call(
        _inv_body,
        out_shape=jax.ShapeDtypeStruct((CROW, D), jnp.float32),
    )(cnt)


def _tform_body(h_ref, w_ref, root_ref, xw_ref, hr_ref):
    h = h_ref[...]
    for r in range(R):
        xw_ref[:, r, :] = jnp.dot(h, w_ref[r], preferred_element_type=jnp.float32)
    hr_ref[...] = jnp.dot(h, root_ref[...], preferred_element_type=jnp.float32)


def _tc_transform(h, W, root):
    return pl.pallas_call(
        _tform_body,
        grid=(N // _BN,),
        in_specs=[
            pl.BlockSpec((_BN, D), lambda i: (i, 0)),
            pl.BlockSpec((R, D, D), lambda i: (0, 0, 0)),
            pl.BlockSpec((D, D), lambda i: (0, 0)),
        ],
        out_specs=[
            pl.BlockSpec((_BN, R, D), lambda i: (i, 0, 0)),
            pl.BlockSpec((_BN, D), lambda i: (i, 0)),
        ],
        out_shape=[
            jax.ShapeDtypeStruct((N, R, D), jnp.float32),
            jax.ShapeDtypeStruct((N, D), jnp.float32),
        ],
    )(h, W, root)


def _mid_body(a_ref, pr_ref, b_ref, w_ref, root_ref, xw_ref, hr_ref):
    h = jax.nn.relu(a_ref[0] + a_ref[1] + pr_ref[...] + b_ref[...])
    for r in range(R):
        xw_ref[:, r, :] = jnp.dot(h, w_ref[r], preferred_element_type=jnp.float32)
    hr_ref[...] = jnp.dot(h, root_ref[...], preferred_element_type=jnp.float32)


def _tc_mid(acc, prevroot, b, W, root):
    return pl.pallas_call(
        _mid_body,
        grid=(N // _BN,),
        in_specs=[
            pl.BlockSpec((NC, _BN, D), lambda i: (0, i, 0)),
            pl.BlockSpec((_BN, D), lambda i: (i, 0)),
            pl.BlockSpec((1, D), lambda i: (0, 0)),
            pl.BlockSpec((R, D, D), lambda i: (0, 0, 0)),
            pl.BlockSpec((D, D), lambda i: (0, 0)),
        ],
        out_specs=[
            pl.BlockSpec((_BN, R, D), lambda i: (i, 0, 0)),
            pl.BlockSpec((_BN, D), lambda i: (i, 0)),
        ],
        out_shape=[
            jax.ShapeDtypeStruct((N, R, D), jnp.float32),
            jax.ShapeDtypeStruct((N, D), jnp.float32),
        ],
    )(acc, prevroot, b, W, root)


def _final_body(a_ref, hr_ref, b_ref, o_ref):
    o_ref[...] = jax.nn.relu(a_ref[0] + a_ref[1] + hr_ref[...] + b_ref[...])


def _tc_final(acc, hroot, b):
    return pl.pallas_call(
        _final_body,
        grid=(N // _BN,),
        in_specs=[
            pl.BlockSpec((NC, _BN, D), lambda i: (0, i, 0)),
            pl.BlockSpec((_BN, D), lambda i: (i, 0)),
            pl.BlockSpec((1, D), lambda i: (0, 0)),
        ],
        out_specs=pl.BlockSpec((_BN, D), lambda i: (i, 0)),
        out_shape=jax.ShapeDtypeStruct((N, D), jnp.float32),
    )(acc, hroot, b)


def kernel(x, edge_index, edge_type, W1, root1, b1, W2, root2, b2):
    pad = EPAD - E

    def _prep_edges(arr, padvals):
        a = jnp.concatenate([arr.astype(jnp.int32), padvals]).reshape(NCH, CH)
        return a.reshape(CPT, NW, CH).transpose(1, 0, 2).reshape(NCH, CH)

    pad_dst = DUMP + (jnp.arange(pad, dtype=jnp.int32) % (NPAD - N))
    src_p = _prep_edges(edge_index[0], jnp.zeros((pad,), jnp.int32))
    dst_p = _prep_edges(edge_index[1], pad_dst)
    et_p = _prep_edges(edge_type, jnp.zeros((pad,), jnp.int32))
    zacc = jnp.zeros((ACC_PT, D), jnp.float32)

    cnt = _sc_count(dst_p, et_p)
    inv = _tc_inv(cnt)
    scale_p, gidx_p = _sc_scale(src_p, dst_p, et_p, inv)

    xw1, xroot1 = _tc_transform(x, W1, root1)
    a1 = _sc_layer(xw1.reshape(NSEG, D), gidx_p, dst_p, scale_p, zacc)
    xw2, hroot2 = _tc_mid(a1[:, :N], xroot1, b1.reshape(1, D), W2, root2)
    a2 = _sc_layer(xw2.reshape(NSEG, D), gidx_p, dst_p, scale_p, zacc)
    return _tc_final(a2[:, :N], hroot2, b2.reshape(1, D))
